# baseline (speedup 1.0000x reference)
import functools

import jax
import jax.numpy as jnp
from jax import lax
from jax.experimental import pallas as pl
from jax.experimental.pallas import tpu as pltpu
from jax.experimental.pallas import tpu_sc as plsc

N = 10000
E = 320000
D = 128
C = 64

NC = 2
NS = 16
NW = NC * NS
EPW = E // NW
CH1 = 80
NCH1 = EPW // CH1
CH2 = 120
NCH2 = -(-EPW // CH2)
EPAD = NW * NCH2 * CH2 - E
RPT = N // NS
RBLK = 2000
L = 16


def _seg_sum_build(with_deg, chunk, nchunk):
    mesh = plsc.VectorSubcoreMesh(core_axis_name="c", subcore_axis_name="s")
    out_type = [jax.ShapeDtypeStruct((NC * N, D), jnp.float32)]
    scratch = [
        pltpu.VMEM_SHARED((N + 8, D), jnp.float32),
        pltpu.VMEM((nchunk, chunk), jnp.int32),
        pltpu.VMEM((nchunk, chunk), jnp.int32),
        pltpu.VMEM((chunk, D), jnp.float32),
        pltpu.VMEM((chunk, D), jnp.float32),
        pltpu.SemaphoreType.DMA,
        pltpu.SemaphoreType.DMA,
        pltpu.SemaphoreType.DMA,
        pltpu.SemaphoreType.DMA,
    ]
    if with_deg:
        out_type.append(
            jax.ShapeDtypeStruct((N // RBLK, NW, RBLK), jnp.float32))
        scratch.append(pltpu.VMEM((N,), jnp.float32))

    @functools.partial(
        pl.kernel,
        mesh=mesh,
        out_type=out_type,
        compiler_params=pltpu.CompilerParams(use_tc_tiling_on_sc=False,
                                             needs_layout_passes=False),
        scratch_types=scratch,
    )
    def seg_sum(src_hbm, dst_hbm, table_hbm, zeros_hbm, out_hbm, *rest):
        if with_deg:
            (deg_hbm, acc_sh, sidx_v, didx_v, rows0_v, rows1_v,
             sem_g0, sem_g1, sem_s0, sem_s1, deg_v) = rest
        else:
            (acc_sh, sidx_v, didx_v, rows0_v, rows1_v,
             sem_g0, sem_g1, sem_s0, sem_s1) = rest
        c = lax.axis_index("c")
        s = lax.axis_index("s")
        wid = s * NC + c

        r0 = s * RPT
        pltpu.sync_copy(zeros_hbm.at[pl.ds(r0, RPT)],
                        acc_sh.at[pl.ds(r0, RPT)])
        pltpu.sync_copy(src_hbm.at[wid], sidx_v)
        pltpu.sync_copy(dst_hbm.at[wid], didx_v)
        if with_deg:
            zv = jnp.zeros((L,), jnp.float32)

            def zero_body(i, carry):
                deg_v[pl.ds(i * L, L)] = zv
                return carry
            lax.fori_loop(0, N // L, zero_body, 0)
        plsc.subcore_barrier()

        def gather(i, rows_v, sem):
            pltpu.async_copy(table_hbm.at[sidx_v.at[i]], rows_v, sem)

        def gather_wait(i, rows_v, sem):
            pltpu.make_async_copy(table_hbm.at[sidx_v.at[i]], rows_v,
                                  sem).wait()

        def scatter(i, rows_v, sem):
            pltpu.async_copy(rows_v, acc_sh.at[didx_v.at[i]], sem, add=True)

        def scatter_wait(i, rows_v, sem):
            pltpu.make_async_copy(rows_v, acc_sh.at[didx_v.at[i]],
                                  sem).wait()

        if with_deg:
            ones = jnp.ones((L,), jnp.float32)

            def count(i):
                for j in range(chunk // L):
                    dv = didx_v[i, pl.ds(j * L, L)]
                    plsc.addupdate_scatter(deg_v, [dv], ones)
        else:
            def count(i):
                pass

        gather(0, rows0_v, sem_g0)
        gather_wait(0, rows0_v, sem_g0)
        scatter(0, rows0_v, sem_s0)
        gather(1, rows1_v, sem_g1)
        count(0)

        def pair(p, carry):
            i0 = 2 * p + 1
            gather_wait(i0, rows1_v, sem_g1)
            scatter(i0, rows1_v, sem_s1)
            scatter_wait(i0 - 1, rows0_v, sem_s0)
            gather(i0 + 1, rows0_v, sem_g0)
            count(i0)
            gather_wait(i0 + 1, rows0_v, sem_g0)
            scatter(i0 + 1, rows0_v, sem_s0)
            scatter_wait(i0, rows1_v, sem_s1)
            gather(i0 + 2, rows1_v, sem_g1)
            count(i0 + 1)
            return carry

        lax.fori_loop(0, (nchunk - 2) // 2, pair, 0)

        last = nchunk - 1
        if nchunk % 2 == 0:
            gather_wait(last, rows1_v, sem_g1)
            scatter(last, rows1_v, sem_s1)
            count(last)
            scatter_wait(last - 1, rows0_v, sem_s0)
            scatter_wait(last, rows1_v, sem_s1)
        else:
            gather_wait(last - 1, rows1_v, sem_g1)
            scatter(last - 1, rows1_v, sem_s1)
            scatter_wait(last - 2, rows0_v, sem_s0)
            gather(last, rows0_v, sem_g0)
            count(last - 1)
            gather_wait(last, rows0_v, sem_g0)
            scatter(last, rows0_v, sem_s0)
            count(last)
            scatter_wait(last - 1, rows1_v, sem_s1)
            scatter_wait(last, rows0_v, sem_s0)

        plsc.subcore_barrier()
        pltpu.sync_copy(acc_sh.at[pl.ds(r0, RPT)],
                        out_hbm.at[pl.ds(c * N + r0, RPT)])
        if with_deg:
            for k in range(N // RBLK):
                pltpu.sync_copy(deg_v.at[pl.ds(k * RBLK, RBLK)],
                                deg_hbm.at[k].at[wid])

    return seg_sum


_seg_sum1 = _seg_sum_build(True, CH1, NCH1)
_seg_sum2 = _seg_sum_build(False, CH2, NCH2)


def _pre1_body(x_ref, wr_ref, b_ref, o_ref):
    o_ref[...] = (jnp.dot(x_ref[...], wr_ref[...],
                          preferred_element_type=jnp.float32) + b_ref[...])


def _post1_body(p_ref, dp_ref, xr_ref, wl_ref, wr2_ref, b2_ref,
                h_ref, hr_ref, deg_ref):
    agg = p_ref[0] + p_ref[1]
    deg = jnp.sum(dp_ref[0], axis=0).reshape(RBLK, 1)
    mean = agg * (1.0 / jnp.maximum(deg, 1.0))
    h = jnp.dot(mean, wl_ref[...], preferred_element_type=jnp.float32)
    h = jnp.maximum(h + xr_ref[...], 0.0)
    h_ref[...] = h
    hr_ref[...] = (jnp.dot(h, wr2_ref[...],
                           preferred_element_type=jnp.float32) + b2_ref[...])
    deg_ref[...] = jnp.broadcast_to(deg, (RBLK, 8))


def _post2_body(p_ref, hr_ref, deg_ref, wl_ref,
                w1_ref, bl1_ref, w2_ref, bl2_ref, o_ref):
    agg = p_ref[0] + p_ref[1]
    deg = deg_ref[...][:, :1]
    mean = agg * (1.0 / jnp.maximum(deg, 1.0))
    h = jnp.dot(mean, wl_ref[...], preferred_element_type=jnp.float32)
    h = jnp.maximum(h + hr_ref[...], 0.0)
    h = jnp.maximum(
        jnp.dot(h, w1_ref[...], preferred_element_type=jnp.float32)
        + bl1_ref[...], 0.0)
    o_ref[...] = (jnp.dot(h, w2_ref[...], preferred_element_type=jnp.float32)
                  + bl2_ref[...])


_mat_spec = pl.BlockSpec((D, D), lambda i: (0, 0))
_vec_spec = pl.BlockSpec((1, D), lambda i: (0, 0))
_h_spec = pl.BlockSpec((RBLK, D), lambda i: (i, 0))
_deg_spec = pl.BlockSpec((RBLK, 8), lambda i: (i, 0))
_p_spec = pl.BlockSpec((NC, RBLK, D), lambda i: (0, i, 0))

_pre1 = pl.pallas_call(
    _pre1_body,
    grid=(N // RBLK,),
    in_specs=[_h_spec, _mat_spec, _vec_spec],
    out_specs=_h_spec,
    out_shape=jax.ShapeDtypeStruct((N, D), jnp.float32),
)

_post1 = pl.pallas_call(
    _post1_body,
    grid=(N // RBLK,),
    in_specs=[_p_spec,
              pl.BlockSpec((1, NW, RBLK), lambda i: (i, 0, 0)),
              _h_spec,
              _mat_spec, _mat_spec, _vec_spec],
    out_specs=[_h_spec, _h_spec, _deg_spec],
    out_shape=[jax.ShapeDtypeStruct((N, D), jnp.float32),
               jax.ShapeDtypeStruct((N, D), jnp.float32),
               jax.ShapeDtypeStruct((N, 8), jnp.float32)],
)

_post2 = pl.pallas_call(
    _post2_body,
    grid=(N // RBLK,),
    in_specs=[_p_spec,
              _h_spec, _deg_spec,
              _mat_spec,
              _mat_spec, _vec_spec,
              pl.BlockSpec((D, C), lambda i: (0, 0)),
              pl.BlockSpec((1, C), lambda i: (0, 0))],
    out_specs=pl.BlockSpec((RBLK, C), lambda i: (i, 0)),
    out_shape=jax.ShapeDtypeStruct((N, C), jnp.float32),
)


def kernel(x, edge_index, Wl1, Wr1, b1, Wl2, Wr2, b2, W1, bl1, W2, bl2):
    src = edge_index[0]
    dst = edge_index[1]
    src1 = src.reshape(NW, NCH1, CH1)
    dst1 = dst.reshape(NW, NCH1, CH1)
    src2 = jnp.concatenate(
        [src, jnp.zeros((EPAD,), jnp.int32)]).reshape(NW, NCH2, CH2)
    dst2 = jnp.concatenate(
        [dst, jnp.full((EPAD,), N, jnp.int32)]).reshape(NW, NCH2, CH2)
    zeros = jnp.zeros((N, D), jnp.float32)

    xr = _pre1(x, Wr1, b1.reshape(1, D))
    p1, dp = _seg_sum1(src1, dst1, x, zeros)
    h1, hr, deg = _post1(p1.reshape(NC, N, D), dp, xr,
                         Wl1, Wr2, b2.reshape(1, D))
    p2, = _seg_sum2(src2, dst2, h1, zeros)
    out = _post2(p2.reshape(NC, N, D), hr, deg, Wl2,
                 W1, bl1.reshape(1, D), W2, bl2.reshape(1, C))
    return out


# speedup vs baseline: 1.2382x; 1.2382x over previous
import functools

import jax
import jax.numpy as jnp
from jax import lax
from jax.experimental import pallas as pl
from jax.experimental.pallas import tpu as pltpu
from jax.experimental.pallas import tpu_sc as plsc

N = 10000
E = 320000
D = 128
C = 64

NC = 2
NS = 16
NW = NC * NS
EPW = E // NW
CH1 = 80
NCH1 = EPW // CH1
CH2 = 80
NCH2 = EPW // CH2
RPT = N // NS
RBLK = 2000
L = 16


def _seg_sum_build(with_deg, chunk, nchunk):
    mesh = plsc.VectorSubcoreMesh(core_axis_name="c", subcore_axis_name="s")
    out_type = [jax.ShapeDtypeStruct((NC * N, D), jnp.float32)]
    scratch = [
        pltpu.VMEM_SHARED((N + 8, D), jnp.float32),
        pltpu.VMEM((nchunk, chunk), jnp.int32),
        pltpu.VMEM((nchunk, chunk), jnp.int32),
        pltpu.VMEM((chunk, D), jnp.float32),
        pltpu.VMEM((chunk, D), jnp.float32),
        pltpu.SemaphoreType.DMA,
        pltpu.SemaphoreType.DMA,
        pltpu.SemaphoreType.DMA,
        pltpu.SemaphoreType.DMA,
    ]
    if with_deg:
        out_type.append(
            jax.ShapeDtypeStruct((N // RBLK, NW, RBLK), jnp.float32))
        scratch.append(pltpu.VMEM((N,), jnp.float32))

    @functools.partial(
        pl.kernel,
        mesh=mesh,
        out_type=out_type,
        compiler_params=pltpu.CompilerParams(use_tc_tiling_on_sc=False,
                                             needs_layout_passes=False),
        scratch_types=scratch,
    )
    def seg_sum(src_hbm, dst_hbm, table_hbm, zeros_hbm, out_hbm, *rest):
        if with_deg:
            (deg_hbm, acc_sh, sidx_v, didx_v, rows0_v, rows1_v,
             sem_g0, sem_g1, sem_s0, sem_s1, deg_v) = rest
        else:
            (acc_sh, sidx_v, didx_v, rows0_v, rows1_v,
             sem_g0, sem_g1, sem_s0, sem_s1) = rest
        c = lax.axis_index("c")
        s = lax.axis_index("s")
        wid = s * NC + c

        r0 = s * RPT
        pltpu.sync_copy(zeros_hbm.at[pl.ds(r0, RPT)],
                        acc_sh.at[pl.ds(r0, RPT)])
        pltpu.sync_copy(src_hbm.at[wid], sidx_v)
        pltpu.sync_copy(dst_hbm.at[wid], didx_v)
        if with_deg:
            zv = jnp.zeros((L,), jnp.float32)

            def zero_body(i, carry):
                deg_v[pl.ds(i * L, L)] = zv
                return carry
            lax.fori_loop(0, N // L, zero_body, 0)
        plsc.subcore_barrier()

        def gather(i, rows_v, sem):
            pltpu.async_copy(table_hbm.at[sidx_v.at[i]], rows_v, sem)

        def gather_wait(i, rows_v, sem):
            pltpu.make_async_copy(table_hbm.at[sidx_v.at[i]], rows_v,
                                  sem).wait()

        def scatter(i, rows_v, sem):
            pltpu.async_copy(rows_v, acc_sh.at[didx_v.at[i]], sem, add=True)

        def scatter_wait(i, rows_v, sem):
            pltpu.make_async_copy(rows_v, acc_sh.at[didx_v.at[i]],
                                  sem).wait()

        if with_deg:
            ones = jnp.ones((L,), jnp.float32)

            def count(i):
                for j in range(chunk // L):
                    dv = didx_v[i, pl.ds(j * L, L)]
                    plsc.addupdate_scatter(deg_v, [dv], ones)
        else:
            def count(i):
                pass

        gather(0, rows0_v, sem_g0)
        gather_wait(0, rows0_v, sem_g0)
        scatter(0, rows0_v, sem_s0)
        gather(1, rows1_v, sem_g1)
        count(0)

        def pair(p, carry):
            i0 = 2 * p + 1
            gather_wait(i0, rows1_v, sem_g1)
            scatter(i0, rows1_v, sem_s1)
            scatter_wait(i0 - 1, rows0_v, sem_s0)
            gather(i0 + 1, rows0_v, sem_g0)
            count(i0)
            gather_wait(i0 + 1, rows0_v, sem_g0)
            scatter(i0 + 1, rows0_v, sem_s0)
            scatter_wait(i0, rows1_v, sem_s1)
            gather(i0 + 2, rows1_v, sem_g1)
            count(i0 + 1)
            return carry

        lax.fori_loop(0, (nchunk - 2) // 2, pair, 0)

        last = nchunk - 1
        if nchunk % 2 == 0:
            gather_wait(last, rows1_v, sem_g1)
            scatter(last, rows1_v, sem_s1)
            count(last)
            scatter_wait(last - 1, rows0_v, sem_s0)
            scatter_wait(last, rows1_v, sem_s1)
        else:
            gather_wait(last - 1, rows1_v, sem_g1)
            scatter(last - 1, rows1_v, sem_s1)
            scatter_wait(last - 2, rows0_v, sem_s0)
            gather(last, rows0_v, sem_g0)
            count(last - 1)
            gather_wait(last, rows0_v, sem_g0)
            scatter(last, rows0_v, sem_s0)
            count(last)
            scatter_wait(last - 1, rows1_v, sem_s1)
            scatter_wait(last, rows0_v, sem_s0)

        plsc.subcore_barrier()
        pltpu.sync_copy(acc_sh.at[pl.ds(r0, RPT)],
                        out_hbm.at[pl.ds(c * N + r0, RPT)])
        if with_deg:
            for k in range(N // RBLK):
                pltpu.sync_copy(deg_v.at[pl.ds(k * RBLK, RBLK)],
                                deg_hbm.at[k].at[wid])

    return seg_sum


_seg_sum1 = _seg_sum_build(True, CH1, NCH1)
_seg_sum2 = _seg_sum_build(False, CH2, NCH2)


def _pre1_body(x_ref, wr_ref, b_ref, o_ref):
    o_ref[...] = (jnp.dot(x_ref[...], wr_ref[...],
                          preferred_element_type=jnp.float32) + b_ref[...])


def _post1_body(p_ref, dp_ref, xr_ref, wl_ref, wr2_ref, b2_ref,
                h_ref, hr_ref, deg_ref):
    agg = p_ref[0] + p_ref[1]
    deg = jnp.sum(dp_ref[0], axis=0).reshape(RBLK, 1)
    mean = agg * (1.0 / jnp.maximum(deg, 1.0))
    h = jnp.dot(mean, wl_ref[...], preferred_element_type=jnp.float32)
    h = jnp.maximum(h + xr_ref[...], 0.0)
    h_ref[...] = h
    hr_ref[...] = (jnp.dot(h, wr2_ref[...],
                           preferred_element_type=jnp.float32) + b2_ref[...])
    deg_ref[...] = jnp.broadcast_to(deg, (RBLK, 8))


def _post2_body(p_ref, hr_ref, deg_ref, wl_ref,
                w1_ref, bl1_ref, w2_ref, bl2_ref, o_ref):
    agg = p_ref[0] + p_ref[1]
    deg = deg_ref[...][:, :1]
    mean = agg * (1.0 / jnp.maximum(deg, 1.0))
    h = jnp.dot(mean, wl_ref[...], preferred_element_type=jnp.float32)
    h = jnp.maximum(h + hr_ref[...], 0.0)
    h = jnp.maximum(
        jnp.dot(h, w1_ref[...], preferred_element_type=jnp.float32)
        + bl1_ref[...], 0.0)
    o_ref[...] = (jnp.dot(h, w2_ref[...], preferred_element_type=jnp.float32)
                  + bl2_ref[...])


_mat_spec = pl.BlockSpec((D, D), lambda i: (0, 0))
_vec_spec = pl.BlockSpec((1, D), lambda i: (0, 0))
_h_spec = pl.BlockSpec((RBLK, D), lambda i: (i, 0))
_deg_spec = pl.BlockSpec((RBLK, 8), lambda i: (i, 0))
_p_spec = pl.BlockSpec((NC, RBLK, D), lambda i: (0, i, 0))

_pre1 = pl.pallas_call(
    _pre1_body,
    grid=(N // RBLK,),
    in_specs=[_h_spec, _mat_spec, _vec_spec],
    out_specs=_h_spec,
    out_shape=jax.ShapeDtypeStruct((N, D), jnp.float32),
)

_post1 = pl.pallas_call(
    _post1_body,
    grid=(N // RBLK,),
    in_specs=[_p_spec,
              pl.BlockSpec((1, NW, RBLK), lambda i: (i, 0, 0)),
              _h_spec,
              _mat_spec, _mat_spec, _vec_spec],
    out_specs=[_h_spec, _h_spec, _deg_spec],
    out_shape=[jax.ShapeDtypeStruct((N, D), jnp.float32),
               jax.ShapeDtypeStruct((N, D), jnp.float32),
               jax.ShapeDtypeStruct((N, 8), jnp.float32)],
)

_post2 = pl.pallas_call(
    _post2_body,
    grid=(N // RBLK,),
    in_specs=[_p_spec,
              _h_spec, _deg_spec,
              _mat_spec,
              _mat_spec, _vec_spec,
              pl.BlockSpec((D, C), lambda i: (0, 0)),
              pl.BlockSpec((1, C), lambda i: (0, 0))],
    out_specs=pl.BlockSpec((RBLK, C), lambda i: (i, 0)),
    out_shape=jax.ShapeDtypeStruct((N, C), jnp.float32),
)


def kernel(x, edge_index, Wl1, Wr1, b1, Wl2, Wr2, b2, W1, bl1, W2, bl2):
    src = edge_index[0]
    dst = edge_index[1]
    src1 = src.reshape(NW, NCH1, CH1)
    dst1 = dst.reshape(NW, NCH1, CH1)
    src2 = src.reshape(NW, NCH2, CH2)
    dst2 = dst.reshape(NW, NCH2, CH2)
    zeros = jnp.zeros((N, D), jnp.float32)

    xr = _pre1(x, Wr1, b1.reshape(1, D))
    p1, dp = _seg_sum1(src1, dst1, x, zeros)
    h1, hr, deg = _post1(p1.reshape(NC, N, D), dp, xr,
                         Wl1, Wr2, b2.reshape(1, D))
    p2, = _seg_sum2(src2, dst2, h1, zeros)
    out = _post2(p2.reshape(NC, N, D), hr, deg, Wl2,
                 W1, bl1.reshape(1, D), W2, bl2.reshape(1, C))
    return out


# speedup vs baseline: 1.2402x; 1.0016x over previous
import functools

import jax
import jax.numpy as jnp
from jax import lax
from jax.experimental import pallas as pl
from jax.experimental.pallas import tpu as pltpu
from jax.experimental.pallas import tpu_sc as plsc

N = 10000
E = 320000
D = 128
C = 64

NC = 2
NS = 16
NW = NC * NS
EPW = E // NW
CH1 = 80
NCH1 = EPW // CH1
CH2 = 80
NCH2 = EPW // CH2
RPT = N // NS
RBLK = 2000
L = 16


def _seg_sum_build(with_deg, chunk, nchunk):
    mesh = plsc.VectorSubcoreMesh(core_axis_name="c", subcore_axis_name="s")
    out_type = [jax.ShapeDtypeStruct((NC * N, D), jnp.float32)]
    scratch = [
        pltpu.VMEM_SHARED((N, D), jnp.float32),
        pltpu.VMEM((3, chunk), jnp.int32),
        pltpu.VMEM((3, chunk), jnp.int32),
        pltpu.VMEM((3, chunk, D), jnp.float32),
        pltpu.SemaphoreType.DMA,
        pltpu.SemaphoreType.DMA,
        pltpu.SemaphoreType.DMA,
        pltpu.SemaphoreType.DMA,
        pltpu.SemaphoreType.DMA,
        pltpu.SemaphoreType.DMA,
        pltpu.SemaphoreType.DMA,
    ]
    if with_deg:
        out_type.append(
            jax.ShapeDtypeStruct((N // RBLK, NW, RBLK), jnp.float32))
        scratch.append(pltpu.VMEM((N,), jnp.float32))

    @functools.partial(
        pl.kernel,
        mesh=mesh,
        out_type=out_type,
        compiler_params=pltpu.CompilerParams(use_tc_tiling_on_sc=False,
                                             needs_layout_passes=False),
        scratch_types=scratch,
    )
    def seg_sum(src_hbm, dst_hbm, table_hbm, zeros_hbm, out_hbm, *rest):
        if with_deg:
            (deg_hbm, acc_sh, sidx_v, didx_v, rows_v, sem_g,
             sem_s0, sem_s1, sem_s2, sem_i0, sem_i1, sem_i2, deg_v) = rest
        else:
            (acc_sh, sidx_v, didx_v, rows_v, sem_g,
             sem_s0, sem_s1, sem_s2, sem_i0, sem_i1, sem_i2) = rest
        sem_s = [sem_s0, sem_s1, sem_s2]
        sem_i = [sem_i0, sem_i1, sem_i2]
        c = lax.axis_index("c")
        s = lax.axis_index("s")
        wid = s * NC + c

        r0 = s * RPT
        pltpu.sync_copy(zeros_hbm.at[pl.ds(r0, RPT)],
                        acc_sh.at[pl.ds(r0, RPT)])
        if with_deg:
            zv = jnp.zeros((L,), jnp.float32)

            def zero_body(i, carry):
                deg_v[pl.ds(i * L, L)] = zv
                return carry
            lax.fori_loop(0, N // L, zero_body, 0)
        plsc.subcore_barrier()

        def idx_load(i, b):
            pltpu.async_copy(src_hbm.at[wid].at[i], sidx_v.at[b], sem_i[b])
            pltpu.async_copy(dst_hbm.at[wid].at[i], didx_v.at[b], sem_i[b])

        def idx_wait(i, b):
            pltpu.make_async_copy(src_hbm.at[wid].at[i], sidx_v.at[b],
                                  sem_i[b]).wait()
            pltpu.make_async_copy(dst_hbm.at[wid].at[i], didx_v.at[b],
                                  sem_i[b]).wait()

        def gather(b):
            pltpu.async_copy(table_hbm.at[sidx_v.at[b]], rows_v.at[b],
                             sem_g)

        def gather_wait(b):
            pltpu.make_async_copy(table_hbm.at[sidx_v.at[b]], rows_v.at[b],
                                  sem_g).wait()

        def scatter(b):
            pltpu.async_copy(rows_v.at[b], acc_sh.at[didx_v.at[b]],
                             sem_s[b], add=True)

        def scatter_wait(b):
            pltpu.make_async_copy(rows_v.at[b], acc_sh.at[didx_v.at[b]],
                                  sem_s[b]).wait()

        if with_deg:
            ones = jnp.ones((L,), jnp.float32)

            def count(b):
                for j in range(chunk // L):
                    dv = didx_v[b, pl.ds(j * L, L)]
                    plsc.addupdate_scatter(deg_v, [dv], ones)
        else:
            def count(b):
                pass

        def step(i, b, do_sw, il_i, iw_g_i):
            gather_wait(b)
            scatter(b)
            if do_sw:
                scatter_wait((b + 2) % 3)
            if il_i is not None:
                idx_load(il_i, (b + 2) % 3)
            if iw_g_i is not None:
                idx_wait(iw_g_i, (b + 1) % 3)
                gather((b + 1) % 3)
            count(b)

        idx_load(0, 0)
        idx_load(1, 1)
        idx_wait(0, 0)
        gather(0)
        step(0, 0, False, 2, 1)

        def tri(t, carry):
            i0 = 3 * t + 1
            step(i0, 1, True, i0 + 2, i0 + 1)
            step(i0 + 1, 2, True, i0 + 3, i0 + 2)
            step(i0 + 2, 0, True, i0 + 4, i0 + 3)
            return carry

        lax.fori_loop(0, (nchunk - 5) // 3, tri, 0)

        last = nchunk - 1
        b0 = (last - 3) % 3
        step(last - 3, b0, True, last - 1, last - 2)
        step(last - 2, (b0 + 1) % 3, True, last, last - 1)
        step(last - 1, (b0 + 2) % 3, True, None, last)
        step(last, b0, True, None, None)
        scatter_wait(b0)

        plsc.subcore_barrier()
        pltpu.sync_copy(acc_sh.at[pl.ds(r0, RPT)],
                        out_hbm.at[pl.ds(c * N + r0, RPT)])
        if with_deg:
            for k in range(N // RBLK):
                pltpu.sync_copy(deg_v.at[pl.ds(k * RBLK, RBLK)],
                                deg_hbm.at[k].at[wid])

    return seg_sum


_seg_sum1 = _seg_sum_build(True, CH1, NCH1)
_seg_sum2 = _seg_sum_build(False, CH2, NCH2)


def _pre1_body(x_ref, wr_ref, b_ref, o_ref):
    o_ref[...] = (jnp.dot(x_ref[...], wr_ref[...],
                          preferred_element_type=jnp.float32) + b_ref[...])


def _post1_body(p_ref, dp_ref, xr_ref, wl_ref, wr2_ref, b2_ref,
                h_ref, hr_ref, deg_ref):
    agg = p_ref[0] + p_ref[1]
    deg = jnp.sum(dp_ref[0], axis=0).reshape(RBLK, 1)
    mean = agg * (1.0 / jnp.maximum(deg, 1.0))
    h = jnp.dot(mean, wl_ref[...], preferred_element_type=jnp.float32)
    h = jnp.maximum(h + xr_ref[...], 0.0)
    h_ref[...] = h
    hr_ref[...] = (jnp.dot(h, wr2_ref[...],
                           preferred_element_type=jnp.float32) + b2_ref[...])
    deg_ref[...] = jnp.broadcast_to(deg, (RBLK, 8))


def _post2_body(p_ref, hr_ref, deg_ref, wl_ref,
                w1_ref, bl1_ref, w2_ref, bl2_ref, o_ref):
    agg = p_ref[0] + p_ref[1]
    deg = deg_ref[...][:, :1]
    mean = agg * (1.0 / jnp.maximum(deg, 1.0))
    h = jnp.dot(mean, wl_ref[...], preferred_element_type=jnp.float32)
    h = jnp.maximum(h + hr_ref[...], 0.0)
    h = jnp.maximum(
        jnp.dot(h, w1_ref[...], preferred_element_type=jnp.float32)
        + bl1_ref[...], 0.0)
    o_ref[...] = (jnp.dot(h, w2_ref[...], preferred_element_type=jnp.float32)
                  + bl2_ref[...])


_mat_spec = pl.BlockSpec((D, D), lambda i: (0, 0))
_vec_spec = pl.BlockSpec((1, D), lambda i: (0, 0))
_h_spec = pl.BlockSpec((RBLK, D), lambda i: (i, 0))
_deg_spec = pl.BlockSpec((RBLK, 8), lambda i: (i, 0))
_p_spec = pl.BlockSpec((NC, RBLK, D), lambda i: (0, i, 0))

_pre1 = pl.pallas_call(
    _pre1_body,
    grid=(N // RBLK,),
    in_specs=[_h_spec, _mat_spec, _vec_spec],
    out_specs=_h_spec,
    out_shape=jax.ShapeDtypeStruct((N, D), jnp.float32),
)

_post1 = pl.pallas_call(
    _post1_body,
    grid=(N // RBLK,),
    in_specs=[_p_spec,
              pl.BlockSpec((1, NW, RBLK), lambda i: (i, 0, 0)),
              _h_spec,
              _mat_spec, _mat_spec, _vec_spec],
    out_specs=[_h_spec, _h_spec, _deg_spec],
    out_shape=[jax.ShapeDtypeStruct((N, D), jnp.float32),
               jax.ShapeDtypeStruct((N, D), jnp.float32),
               jax.ShapeDtypeStruct((N, 8), jnp.float32)],
)

_post2 = pl.pallas_call(
    _post2_body,
    grid=(N // RBLK,),
    in_specs=[_p_spec,
              _h_spec, _deg_spec,
              _mat_spec,
              _mat_spec, _vec_spec,
              pl.BlockSpec((D, C), lambda i: (0, 0)),
              pl.BlockSpec((1, C), lambda i: (0, 0))],
    out_specs=pl.BlockSpec((RBLK, C), lambda i: (i, 0)),
    out_shape=jax.ShapeDtypeStruct((N, C), jnp.float32),
)


def kernel(x, edge_index, Wl1, Wr1, b1, Wl2, Wr2, b2, W1, bl1, W2, bl2):
    src = edge_index[0]
    dst = edge_index[1]
    src1 = src.reshape(NW, NCH1, CH1)
    dst1 = dst.reshape(NW, NCH1, CH1)
    src2 = src.reshape(NW, NCH2, CH2)
    dst2 = dst.reshape(NW, NCH2, CH2)
    zeros = jnp.zeros((N, D), jnp.float32)

    xr = _pre1(x, Wr1, b1.reshape(1, D))
    p1, dp = _seg_sum1(src1, dst1, x, zeros)
    h1, hr, deg = _post1(p1.reshape(NC, N, D), dp, xr,
                         Wl1, Wr2, b2.reshape(1, D))
    p2, = _seg_sum2(src2, dst2, h1, zeros)
    out = _post2(p2.reshape(NC, N, D), hr, deg, Wl2,
                 W1, bl1.reshape(1, D), W2, bl2.reshape(1, C))
    return out


# speedup vs baseline: 1.2446x; 1.0036x over previous
import functools

import jax
import jax.numpy as jnp
from jax import lax
from jax.experimental import pallas as pl
from jax.experimental.pallas import tpu as pltpu
from jax.experimental.pallas import tpu_sc as plsc

N = 10000
E = 320000
D = 128
C = 64

NC = 2
NS = 16
NW = NC * NS
EPW = E // NW
CH1 = 80
NCH1 = EPW // CH1
CH2 = 80
NCH2 = EPW // CH2
RPT = N // NS
RBLK = 2000
L = 16


def _seg_sum_build(with_deg, chunk, nchunk):
    mesh = plsc.VectorSubcoreMesh(core_axis_name="c", subcore_axis_name="s")
    out_type = [jax.ShapeDtypeStruct((NC * N, D), jnp.float32)]
    scratch = [
        pltpu.VMEM_SHARED((N, D), jnp.float32),
        pltpu.VMEM((3, chunk), jnp.int32),
        pltpu.VMEM((3, chunk), jnp.int32),
        pltpu.VMEM((3, chunk, D), jnp.float32),
        pltpu.SemaphoreType.DMA,
        pltpu.SemaphoreType.DMA,
        pltpu.SemaphoreType.DMA,
        pltpu.SemaphoreType.DMA,
        pltpu.SemaphoreType.DMA,
        pltpu.SemaphoreType.DMA,
        pltpu.SemaphoreType.DMA,
    ]
    if with_deg:
        out_type.append(
            jax.ShapeDtypeStruct((N // RBLK, NW, RBLK), jnp.float32))
        scratch.append(pltpu.VMEM((N,), jnp.float32))

    @functools.partial(
        pl.kernel,
        mesh=mesh,
        out_type=out_type,
        compiler_params=pltpu.CompilerParams(use_tc_tiling_on_sc=False,
                                             needs_layout_passes=False),
        scratch_types=scratch,
    )
    def seg_sum(src_hbm, dst_hbm, table_hbm, zeros_hbm, out_hbm, *rest):
        if with_deg:
            (deg_hbm, acc_sh, sidx_v, didx_v, rows_v, sem_g,
             sem_s0, sem_s1, sem_s2, sem_i0, sem_i1, sem_i2, deg_v) = rest
        else:
            (acc_sh, sidx_v, didx_v, rows_v, sem_g,
             sem_s0, sem_s1, sem_s2, sem_i0, sem_i1, sem_i2) = rest
        sem_s = [sem_s0, sem_s1, sem_s2]
        sem_i = [sem_i0, sem_i1, sem_i2]
        c = lax.axis_index("c")
        s = lax.axis_index("s")
        wid = s * NC + c

        r0 = s * RPT
        pltpu.sync_copy(zeros_hbm.at[pl.ds(r0, RPT)],
                        acc_sh.at[pl.ds(r0, RPT)])
        if with_deg:
            zv = jnp.zeros((L,), jnp.float32)

            def zero_body(i, carry):
                deg_v[pl.ds(i * L, L)] = zv
                return carry
            lax.fori_loop(0, N // L, zero_body, 0)
        plsc.subcore_barrier()

        def idx_load(i, b):
            pltpu.async_copy(src_hbm.at[wid].at[i], sidx_v.at[b], sem_i[b])
            pltpu.async_copy(dst_hbm.at[wid].at[i], didx_v.at[b], sem_i[b])

        def idx_wait(i, b):
            pltpu.make_async_copy(src_hbm.at[wid].at[i], sidx_v.at[b],
                                  sem_i[b]).wait()
            pltpu.make_async_copy(dst_hbm.at[wid].at[i], didx_v.at[b],
                                  sem_i[b]).wait()

        def gather(b):
            pltpu.async_copy(table_hbm.at[sidx_v.at[b]], rows_v.at[b],
                             sem_g)

        def gather_wait(b):
            pltpu.make_async_copy(table_hbm.at[sidx_v.at[b]], rows_v.at[b],
                                  sem_g).wait()

        def scatter(b):
            pltpu.async_copy(rows_v.at[b], acc_sh.at[didx_v.at[b]],
                             sem_s[b], add=True)

        def scatter_wait(b):
            pltpu.make_async_copy(rows_v.at[b], acc_sh.at[didx_v.at[b]],
                                  sem_s[b]).wait()

        if with_deg:
            ones = jnp.ones((L,), jnp.float32)

            def count(b):
                for j in range(chunk // L):
                    dv = didx_v[b, pl.ds(j * L, L)]
                    plsc.addupdate_scatter(deg_v, [dv], ones)
        else:
            def count(b):
                pass

        def step(i, b, do_sw, il_i, iw_g_i):
            gather_wait(b)
            scatter(b)
            if do_sw:
                scatter_wait((b + 2) % 3)
            if il_i is not None:
                idx_load(il_i, (b + 2) % 3)
            if iw_g_i is not None:
                idx_wait(iw_g_i, (b + 1) % 3)
                gather((b + 1) % 3)
            count(b)

        idx_load(0, 0)
        idx_load(1, 1)
        idx_wait(0, 0)
        gather(0)
        step(0, 0, False, 2, 1)

        def tri(t, carry):
            i0 = 3 * t + 1
            step(i0, 1, True, i0 + 2, i0 + 1)
            step(i0 + 1, 2, True, i0 + 3, i0 + 2)
            step(i0 + 2, 0, True, i0 + 4, i0 + 3)
            return carry

        lax.fori_loop(0, (nchunk - 5) // 3, tri, 0)

        last = nchunk - 1
        b0 = (last - 3) % 3
        step(last - 3, b0, True, last - 1, last - 2)
        step(last - 2, (b0 + 1) % 3, True, last, last - 1)
        step(last - 1, (b0 + 2) % 3, True, None, last)
        step(last, b0, True, None, None)
        scatter_wait(b0)

        plsc.subcore_barrier()
        pltpu.sync_copy(acc_sh.at[pl.ds(r0, RPT)],
                        out_hbm.at[pl.ds(c * N + r0, RPT)])
        if with_deg:
            for k in range(N // RBLK):
                pltpu.sync_copy(deg_v.at[pl.ds(k * RBLK, RBLK)],
                                deg_hbm.at[k].at[wid])

    return seg_sum


_seg_sum1 = _seg_sum_build(True, CH1, NCH1)
_seg_sum2 = _seg_sum_build(False, CH2, NCH2)


def _post1_body(p_ref, dp_ref, x_ref, wl_ref, wr_ref, b_ref, wr2_ref, b2_ref,
                h_ref, hr_ref, deg_ref):
    agg = p_ref[0] + p_ref[1]
    deg = jnp.sum(dp_ref[0], axis=0).reshape(RBLK, 1)
    mean = agg * (1.0 / jnp.maximum(deg, 1.0))
    h = jnp.dot(mean, wl_ref[...], preferred_element_type=jnp.float32)
    h = h + jnp.dot(x_ref[...], wr_ref[...],
                    preferred_element_type=jnp.float32) + b_ref[...]
    h = jnp.maximum(h, 0.0)
    h_ref[...] = h
    hr_ref[...] = (jnp.dot(h, wr2_ref[...],
                           preferred_element_type=jnp.float32) + b2_ref[...])
    deg_ref[...] = jnp.broadcast_to(deg, (RBLK, 8))


def _post2_body(p_ref, hr_ref, deg_ref, wl_ref,
                w1_ref, bl1_ref, w2_ref, bl2_ref, o_ref):
    agg = p_ref[0] + p_ref[1]
    deg = deg_ref[...][:, :1]
    mean = agg * (1.0 / jnp.maximum(deg, 1.0))
    h = jnp.dot(mean, wl_ref[...], preferred_element_type=jnp.float32)
    h = jnp.maximum(h + hr_ref[...], 0.0)
    h = jnp.maximum(
        jnp.dot(h, w1_ref[...], preferred_element_type=jnp.float32)
        + bl1_ref[...], 0.0)
    o_ref[...] = (jnp.dot(h, w2_ref[...], preferred_element_type=jnp.float32)
                  + bl2_ref[...])


_mat_spec = pl.BlockSpec((D, D), lambda i: (0, 0))
_vec_spec = pl.BlockSpec((1, D), lambda i: (0, 0))
_h_spec = pl.BlockSpec((RBLK, D), lambda i: (i, 0))
_deg_spec = pl.BlockSpec((RBLK, 8), lambda i: (i, 0))
_p_spec = pl.BlockSpec((NC, RBLK, D), lambda i: (0, i, 0))

_post1 = pl.pallas_call(
    _post1_body,
    grid=(N // RBLK,),
    in_specs=[_p_spec,
              pl.BlockSpec((1, NW, RBLK), lambda i: (i, 0, 0)),
              _h_spec,
              _mat_spec, _mat_spec, _vec_spec, _mat_spec, _vec_spec],
    out_specs=[_h_spec, _h_spec, _deg_spec],
    out_shape=[jax.ShapeDtypeStruct((N, D), jnp.float32),
               jax.ShapeDtypeStruct((N, D), jnp.float32),
               jax.ShapeDtypeStruct((N, 8), jnp.float32)],
)

_post2 = pl.pallas_call(
    _post2_body,
    grid=(N // RBLK,),
    in_specs=[_p_spec,
              _h_spec, _deg_spec,
              _mat_spec,
              _mat_spec, _vec_spec,
              pl.BlockSpec((D, C), lambda i: (0, 0)),
              pl.BlockSpec((1, C), lambda i: (0, 0))],
    out_specs=pl.BlockSpec((RBLK, C), lambda i: (i, 0)),
    out_shape=jax.ShapeDtypeStruct((N, C), jnp.float32),
)


def kernel(x, edge_index, Wl1, Wr1, b1, Wl2, Wr2, b2, W1, bl1, W2, bl2):
    src = edge_index[0]
    dst = edge_index[1]
    src1 = src.reshape(NW, NCH1, CH1)
    dst1 = dst.reshape(NW, NCH1, CH1)
    src2 = src.reshape(NW, NCH2, CH2)
    dst2 = dst.reshape(NW, NCH2, CH2)
    zeros = jnp.zeros((N, D), jnp.float32)

    p1, dp = _seg_sum1(src1, dst1, x, zeros)
    h1, hr, deg = _post1(p1.reshape(NC, N, D), dp, x,
                         Wl1, Wr1, b1.reshape(1, D), Wr2, b2.reshape(1, D))
    p2, = _seg_sum2(src2, dst2, h1, zeros)
    out = _post2(p2.reshape(NC, N, D), hr, deg, Wl2,
                 W1, bl1.reshape(1, D), W2, bl2.reshape(1, C))
    return out
